# Initial kernel scaffold; baseline (speedup 1.0000x reference)
import jax
import jax.numpy as jnp
from jax.experimental import pallas as pl


def kernel(A, X, Wlins, Wmlp):
    raise NotImplementedError("write your pallas kernel here")


# speedup vs baseline: 3.0592x; 3.0592x over previous
import jax
import jax.numpy as jnp
from jax.experimental import pallas as pl

_B, _N, _D = 16, 64, 128
_GB = 2


def _fold_kernel(wlins_ref, wmlp_ref, wbig_ref, wsmall_ref, wsg_ref):
    W = wlins_ref[...]
    M = wmlp_ref[...]

    def f(k):
        return jnp.dot(W[k - 1], M, preferred_element_type=jnp.float32,
                       precision=jax.lax.Precision.HIGHEST)

    wbig_ref[...] = jnp.concatenate([f(10), f(11)], axis=1)
    row_r = jnp.concatenate([f(6), f(7), f(3)], axis=1)
    row_c = jnp.concatenate([f(8), f(9), f(4)], axis=1)
    row_d = jnp.concatenate([f(12), f(13), f(1)], axis=1)
    wsmall_ref[...] = jnp.concatenate([row_r, row_c, row_d], axis=0)
    z = jnp.zeros((_D, _D), jnp.float32)
    top = jnp.concatenate([2.0 * f(14), f(2)], axis=1)
    bot = jnp.concatenate([z, f(5)], axis=1)
    wsg_ref[...] = jnp.concatenate([top, bot], axis=0)


def _main_kernel(x_ref, wbig_ref, wsmall_ref, wsg_ref, out_ref):
    x = x_ref[...]
    wb = wbig_ref[...]
    ws = wsmall_ref[...]
    wg = wsg_ref[...]

    r = jnp.mean(x, axis=2)
    c = jnp.mean(x, axis=1)
    i1 = jax.lax.broadcasted_iota(jnp.int32, (_N, _N), 0)
    i2 = jax.lax.broadcasted_iota(jnp.int32, (_N, _N), 1)
    eye = (i1 == i2).astype(jnp.float32)
    dd = jnp.sum(x * eye[None, :, :, None], axis=2)

    F = jnp.concatenate([r, c, dd], axis=-1)
    S = jax.lax.dot_general(F.reshape(_GB * _N, 3 * _D), ws,
                            (((1,), (0,)), ((), ())),
                            preferred_element_type=jnp.float32)
    S = S.reshape(_GB, _N, 3 * _D)

    s = jnp.mean(dd, axis=1)
    g = jnp.mean(r, axis=1)
    Sg = jnp.concatenate([s, g], axis=-1)
    consts = jnp.sum(Sg[:, :, None] * wg[None, :, :], axis=1)

    YZ = jnp.dot(x.reshape(_GB * _N * _N, _D), wb,
                 preferred_element_type=jnp.float32)
    YZ = YZ.reshape(_GB, _N, _N, 2 * _D)
    acc = YZ[..., :_D] + jnp.swapaxes(YZ[..., _D:], 1, 2)
    acc += S[:, :, None, :_D]
    acc += S[:, None, :, _D:2 * _D]
    acc += consts[:, None, None, :_D]
    diag = S[..., 2 * _D:] + consts[:, None, _D:]
    acc += eye[None, :, :, None] * diag[:, :, None, :]
    out_ref[...] = jnp.maximum(acc, 0.0)


def kernel(A, X, Wlins, Wmlp):
    del A
    wbig, wsmall, wsg = pl.pallas_call(
        _fold_kernel,
        out_shape=(
            jax.ShapeDtypeStruct((_D, 2 * _D), jnp.float32),
            jax.ShapeDtypeStruct((3 * _D, 3 * _D), jnp.float32),
            jax.ShapeDtypeStruct((2 * _D, 2 * _D), jnp.float32),
        ),
    )(Wlins, Wmlp)

    out = pl.pallas_call(
        _main_kernel,
        grid=(_B // _GB,),
        in_specs=[
            pl.BlockSpec((_GB, _N, _N, _D), lambda b: (b, 0, 0, 0)),
            pl.BlockSpec((_D, 2 * _D), lambda b: (0, 0)),
            pl.BlockSpec((3 * _D, 3 * _D), lambda b: (0, 0)),
            pl.BlockSpec((2 * _D, 2 * _D), lambda b: (0, 0)),
        ],
        out_specs=pl.BlockSpec((_GB, _N, _N, _D), lambda b: (b, 0, 0, 0)),
        out_shape=jax.ShapeDtypeStruct((_B, _N, _N, _D), jnp.float32),
    )(X, wbig, wsmall, wsg)
    return out
